# baseline (speedup 1.0000x reference)
import functools

import jax
import jax.numpy as jnp
from jax import lax
from jax.experimental import pallas as pl
from jax.experimental.pallas import tpu as pltpu
from jax.experimental.pallas import tpu_sc as plsc

NUM_RADIAL = 128
NUM_SPH = 8
EMB = 32
OUT_COLS = NUM_SPH * EMB

_NC = 2
_NS = 16
_NW = _NC * _NS
_CHUNK = 128


def _sc_gather(table, idx, t_pad, nc0, nc1):
    mesh = plsc.VectorSubcoreMesh(core_axis_name="c", subcore_axis_name="s")

    @functools.partial(
        pl.kernel,
        mesh=mesh,
        out_type=jax.ShapeDtypeStruct((t_pad, NUM_RADIAL), jnp.float32),
        scratch_types=[
            pltpu.VMEM((_CHUNK,), jnp.int32),
            pltpu.VMEM((_CHUNK,), jnp.int32),
            pltpu.VMEM((_CHUNK, NUM_RADIAL), jnp.float32),
            pltpu.VMEM((_CHUNK, NUM_RADIAL), jnp.float32),
            pltpu.SemaphoreType.DMA,
            pltpu.SemaphoreType.DMA,
            pltpu.SemaphoreType.DMA,
            pltpu.SemaphoreType.DMA,
        ],
    )
    def k(table_hbm, idx_hbm, out_hbm, idx0, idx1, rows0, rows1,
          g0, g1, w0, w1):
        c_ax = lax.axis_index("c")
        s_ax = lax.axis_index("s")
        my_n = jnp.where(c_ax == 0, nc0, nc1)
        base = jnp.where(c_ax == 0, s_ax * nc0, _NS * nc0 + s_ax * nc1)

        def off(c):
            return (base + c) * _CHUNK

        def do_chunk(c, idxb, rowsb, gsem, wsem, drain_first):
            pltpu.sync_copy(idx_hbm.at[pl.ds(off(c), _CHUNK)], idxb)
            if drain_first:
                pltpu.make_async_copy(
                    rowsb, out_hbm.at[pl.ds(off(c), _CHUNK)], wsem).wait()
            pltpu.async_copy(table_hbm.at[idxb], rowsb, gsem).wait()
            pltpu.async_copy(rowsb, out_hbm.at[pl.ds(off(c), _CHUNK)], wsem)

        do_chunk(0, idx0, rows0, g0, w0, False)
        do_chunk(1, idx1, rows1, g1, w1, False)

        def body(j, carry):
            do_chunk(2 * j, idx0, rows0, g0, w0, True)
            do_chunk(2 * j + 1, idx1, rows1, g1, w1, True)
            return carry

        lax.fori_loop(1, my_n // 2, body, 0, unroll=False)
        pltpu.make_async_copy(
            rows0, out_hbm.at[pl.ds(off(my_n - 2), _CHUNK)], w0).wait()
        pltpu.make_async_copy(
            rows1, out_hbm.at[pl.ds(off(my_n - 1), _CHUNK)], w1).wait()

    return k(table, idx)


def _tc_contract(g, sph, w, b_mat, p_mat, t, tile):
    def body(g_ref, s_ref, w_ref, b_ref, p_ref, o_ref):
        h = jnp.dot(g_ref[...], w_ref[...], preferred_element_type=jnp.float32)
        srep = jnp.dot(s_ref[...], b_ref[...], preferred_element_type=jnp.float32)
        o_ref[...] = jnp.dot(h * srep, p_ref[...],
                             preferred_element_type=jnp.float32)

    return pl.pallas_call(
        body,
        grid=(t // tile,),
        in_specs=[
            pl.BlockSpec((tile, NUM_RADIAL), lambda i: (i, 0)),
            pl.BlockSpec((tile, NUM_SPH), lambda i: (i, 0)),
            pl.BlockSpec((NUM_RADIAL, OUT_COLS), lambda i: (0, 0)),
            pl.BlockSpec((NUM_SPH, OUT_COLS), lambda i: (0, 0)),
            pl.BlockSpec((OUT_COLS, EMB), lambda i: (0, 0)),
        ],
        out_specs=pl.BlockSpec((tile, EMB), lambda i: (i, 0)),
        out_shape=jax.ShapeDtypeStruct((t, EMB), jnp.float32),
    )(g, sph, w, b_mat, p_mat)


def kernel(rbf, sph, idx_sph, weight):
    t = idx_sph.shape[0]
    tile = 1000
    n_phase = 2
    tp = t // n_phase

    w = weight.reshape(NUM_RADIAL, OUT_COLS)
    b_mat = jnp.tile(jnp.eye(NUM_SPH, dtype=jnp.float32), (1, EMB))
    p_mat = jnp.repeat(jnp.eye(EMB, dtype=jnp.float32), NUM_SPH, axis=0)

    nchunks = -(-tp // (_NW * _CHUNK))
    nchunks += nchunks % 2
    tp_pad = _NW * nchunks * _CHUNK
    nc0 = max(4, (2 * nchunks * 13 // 20) // 2 * 2)
    nc1 = 2 * nchunks - nc0

    outs = []
    for p in range(n_phase):
        idx_p = jnp.zeros((tp_pad,), jnp.int32).at[:tp].set(
            lax.dynamic_slice_in_dim(idx_sph, p * tp, tp))
        g = _sc_gather(rbf, idx_p, tp_pad, nc0, nc1)
        sph_p = lax.dynamic_slice_in_dim(sph, p * tp, tp)
        outs.append(_tc_contract(g, sph_p, w, b_mat, p_mat, tp, tile))
    return jnp.concatenate(outs, axis=0)


# speedup vs baseline: 1.4343x; 1.4343x over previous
import functools

import jax
import jax.numpy as jnp
from jax import lax
from jax.experimental import pallas as pl
from jax.experimental.pallas import tpu as pltpu
from jax.experimental.pallas import tpu_sc as plsc

NUM_RADIAL = 128
NUM_SPH = 8
EMB = 32
OUT_COLS = NUM_SPH * EMB

_NC = 2
_NS = 16
_NW = _NC * _NS
_CHUNK = 128


def _sc_gather(table, idx, t_pad, nc0, nc1):
    mesh = plsc.VectorSubcoreMesh(core_axis_name="c", subcore_axis_name="s")

    @functools.partial(
        pl.kernel,
        mesh=mesh,
        out_type=jax.ShapeDtypeStruct((t_pad, NUM_RADIAL), jnp.float32),
        scratch_types=[
            pltpu.VMEM((_CHUNK,), jnp.int32),
            pltpu.VMEM((_CHUNK,), jnp.int32),
            pltpu.VMEM((_CHUNK, NUM_RADIAL), jnp.float32),
            pltpu.VMEM((_CHUNK, NUM_RADIAL), jnp.float32),
            pltpu.SemaphoreType.DMA,
            pltpu.SemaphoreType.DMA,
            pltpu.SemaphoreType.DMA,
            pltpu.SemaphoreType.DMA,
            pltpu.SemaphoreType.DMA,
            pltpu.SemaphoreType.DMA,
        ],
    )
    def k(table_hbm, idx_hbm, out_hbm, idx0, idx1, rows0, rows1,
          g0, g1, w0, w1, i0, i1):
        c_ax = lax.axis_index("c")
        s_ax = lax.axis_index("s")
        my_n = jnp.where(c_ax == 0, nc0, nc1)
        base = jnp.where(c_ax == 0, s_ax * nc0, _NS * nc0 + s_ax * nc1)

        def off(c):
            return (base + c) * _CHUNK

        def fetch_idx(c, idxb, isem):
            pltpu.async_copy(idx_hbm.at[pl.ds(off(c), _CHUNK)], idxb, isem)

        def do_chunk(c, idxb, rowsb, gsem, wsem, isem,
                     drain_first, fetch_next):
            pltpu.make_async_copy(
                idx_hbm.at[pl.ds(off(c), _CHUNK)], idxb, isem).wait()
            if drain_first:
                pltpu.make_async_copy(
                    rowsb, out_hbm.at[pl.ds(off(c), _CHUNK)], wsem).wait()
            pltpu.async_copy(table_hbm.at[idxb], rowsb, gsem).wait()
            pltpu.async_copy(rowsb, out_hbm.at[pl.ds(off(c), _CHUNK)], wsem)
            if fetch_next:
                fetch_idx(c + 2, idxb, isem)

        fetch_idx(0, idx0, i0)
        fetch_idx(1, idx1, i1)
        do_chunk(0, idx0, rows0, g0, w0, i0, False, True)
        do_chunk(1, idx1, rows1, g1, w1, i1, False, True)

        def body(j, carry):
            do_chunk(2 * j, idx0, rows0, g0, w0, i0, True, True)
            do_chunk(2 * j + 1, idx1, rows1, g1, w1, i1, True, True)
            return carry

        lax.fori_loop(1, my_n // 2 - 1, body, 0, unroll=False)

        do_chunk(my_n - 2, idx0, rows0, g0, w0, i0, True, False)
        do_chunk(my_n - 1, idx1, rows1, g1, w1, i1, True, False)
        pltpu.make_async_copy(
            rows0, out_hbm.at[pl.ds(off(my_n - 2), _CHUNK)], w0).wait()
        pltpu.make_async_copy(
            rows1, out_hbm.at[pl.ds(off(my_n - 1), _CHUNK)], w1).wait()

    return k(table, idx)


def _tc_contract(g, sph, w, b_mat, p_mat, t, tile):
    def body(g_ref, s_ref, w_ref, b_ref, p_ref, o_ref):
        h = jnp.dot(g_ref[...], w_ref[...], preferred_element_type=jnp.float32)
        srep = jnp.dot(s_ref[...], b_ref[...], preferred_element_type=jnp.float32)
        o_ref[...] = jnp.dot(h * srep, p_ref[...],
                             preferred_element_type=jnp.float32)

    return pl.pallas_call(
        body,
        grid=(t // tile,),
        in_specs=[
            pl.BlockSpec((tile, NUM_RADIAL), lambda i: (i, 0)),
            pl.BlockSpec((tile, NUM_SPH), lambda i: (i, 0)),
            pl.BlockSpec((NUM_RADIAL, OUT_COLS), lambda i: (0, 0)),
            pl.BlockSpec((NUM_SPH, OUT_COLS), lambda i: (0, 0)),
            pl.BlockSpec((OUT_COLS, EMB), lambda i: (0, 0)),
        ],
        out_specs=pl.BlockSpec((tile, EMB), lambda i: (i, 0)),
        out_shape=jax.ShapeDtypeStruct((t, EMB), jnp.float32),
    )(g, sph, w, b_mat, p_mat)


def kernel(rbf, sph, idx_sph, weight):
    t = idx_sph.shape[0]
    tile = 1000
    nchunks = -(-t // (_NW * _CHUNK))
    nchunks += nchunks % 2
    t_pad = _NW * nchunks * _CHUNK
    nc0 = max(4, (2 * nchunks * 13 // 20) // 2 * 2)
    nc1 = 2 * nchunks - nc0

    idx_pad = jnp.zeros((t_pad,), jnp.int32).at[:t].set(idx_sph)

    g = _sc_gather(rbf, idx_pad, t_pad, nc0, nc1)

    w = weight.reshape(NUM_RADIAL, OUT_COLS)
    b_mat = jnp.tile(jnp.eye(NUM_SPH, dtype=jnp.float32), (1, EMB))
    p_mat = jnp.repeat(jnp.eye(EMB, dtype=jnp.float32), NUM_SPH, axis=0)

    return _tc_contract(g, sph, w, b_mat, p_mat, t, tile)


# speedup vs baseline: 1.6209x; 1.1301x over previous
import functools

import jax
import jax.numpy as jnp
from jax import lax
from jax.experimental import pallas as pl
from jax.experimental.pallas import tpu as pltpu
from jax.experimental.pallas import tpu_sc as plsc

NUM_RADIAL = 128
NUM_SPH = 8
EMB = 32
OUT_COLS = NUM_SPH * EMB

_NC = 2
_NS = 16
_NW = _NC * _NS
_CHUNK = 128


def _sc_gather(table, idx, t_pad, nc0, nc1):
    mesh = plsc.VectorSubcoreMesh(core_axis_name="c", subcore_axis_name="s")

    @functools.partial(
        pl.kernel,
        mesh=mesh,
        out_type=jax.ShapeDtypeStruct((t_pad, NUM_RADIAL), jnp.float32),
        scratch_types=[
            pltpu.VMEM((_CHUNK,), jnp.int32),
            pltpu.VMEM((_CHUNK,), jnp.int32),
            pltpu.VMEM((_CHUNK, NUM_RADIAL), jnp.float32),
            pltpu.VMEM((_CHUNK, NUM_RADIAL), jnp.float32),
            pltpu.SemaphoreType.DMA,
            pltpu.SemaphoreType.DMA,
            pltpu.SemaphoreType.DMA,
            pltpu.SemaphoreType.DMA,
            pltpu.SemaphoreType.DMA,
            pltpu.SemaphoreType.DMA,
        ],
    )
    def k(table_hbm, idx_hbm, out_hbm, idx0, idx1, rows0, rows1,
          g0, g1, w0, w1, i0, i1):
        c_ax = lax.axis_index("c")
        s_ax = lax.axis_index("s")
        my_n = jnp.where(c_ax == 0, nc0, nc1)
        base = jnp.where(c_ax == 0, s_ax * nc0, _NS * nc0 + s_ax * nc1)

        def off(c):
            return (base + c) * _CHUNK

        def fetch_idx(c, idxb, isem):
            pltpu.async_copy(idx_hbm.at[pl.ds(off(c), _CHUNK)], idxb, isem)

        def do_chunk(c, idxb, rowsb, gsem, wsem, isem,
                     drain_first, fetch_next):
            pltpu.make_async_copy(
                idx_hbm.at[pl.ds(off(c), _CHUNK)], idxb, isem).wait()
            if drain_first:
                pltpu.make_async_copy(
                    rowsb, out_hbm.at[pl.ds(off(c), _CHUNK)], wsem).wait()
            pltpu.async_copy(table_hbm.at[idxb], rowsb, gsem).wait()
            pltpu.async_copy(rowsb, out_hbm.at[pl.ds(off(c), _CHUNK)], wsem)
            if fetch_next:
                fetch_idx(c + 2, idxb, isem)

        fetch_idx(0, idx0, i0)
        fetch_idx(1, idx1, i1)
        do_chunk(0, idx0, rows0, g0, w0, i0, False, True)
        do_chunk(1, idx1, rows1, g1, w1, i1, False, True)

        def body(j, carry):
            do_chunk(2 * j, idx0, rows0, g0, w0, i0, True, True)
            do_chunk(2 * j + 1, idx1, rows1, g1, w1, i1, True, True)
            return carry

        lax.fori_loop(1, my_n // 2 - 1, body, 0, unroll=False)

        do_chunk(my_n - 2, idx0, rows0, g0, w0, i0, True, False)
        do_chunk(my_n - 1, idx1, rows1, g1, w1, i1, True, False)
        pltpu.make_async_copy(
            rows0, out_hbm.at[pl.ds(off(my_n - 2), _CHUNK)], w0).wait()
        pltpu.make_async_copy(
            rows1, out_hbm.at[pl.ds(off(my_n - 1), _CHUNK)], w1).wait()

    return k(table, idx)


def _tc_contract(g, sph, w, b_mat, p_mat, t, tile):
    def body(g_ref, s_ref, w_ref, b_ref, p_ref, o_ref):
        h = jnp.dot(g_ref[...], w_ref[...], preferred_element_type=jnp.float32)
        srep = jnp.dot(s_ref[...], b_ref[...], preferred_element_type=jnp.float32)
        o_ref[...] = jnp.dot(h * srep, p_ref[...],
                             preferred_element_type=jnp.float32)

    return pl.pallas_call(
        body,
        grid=(t // tile,),
        in_specs=[
            pl.BlockSpec((tile, NUM_RADIAL), lambda i: (i, 0)),
            pl.BlockSpec((tile, NUM_SPH), lambda i: (i, 0)),
            pl.BlockSpec((NUM_RADIAL, OUT_COLS), lambda i: (0, 0)),
            pl.BlockSpec((NUM_SPH, OUT_COLS), lambda i: (0, 0)),
            pl.BlockSpec((OUT_COLS, EMB), lambda i: (0, 0)),
        ],
        out_specs=pl.BlockSpec((tile, EMB), lambda i: (i, 0)),
        out_shape=jax.ShapeDtypeStruct((t, EMB), jnp.float32),
    )(g, sph, w, b_mat, p_mat)


def kernel(rbf, sph, idx_sph, weight):
    t = idx_sph.shape[0]
    tile = 2000
    nchunks = -(-t // (_NW * _CHUNK))
    nchunks += nchunks % 2
    t_pad = _NW * nchunks * _CHUNK
    nc0 = max(4, (2 * nchunks * 13 // 20) // 2 * 2)
    nc1 = 2 * nchunks - nc0

    idx_pad = jnp.zeros((t_pad,), jnp.int32).at[:t].set(idx_sph)

    g = _sc_gather(rbf, idx_pad, t_pad, nc0, nc1)

    w = weight.reshape(NUM_RADIAL, OUT_COLS)
    b_mat = jnp.tile(jnp.eye(NUM_SPH, dtype=jnp.float32), (1, EMB))
    p_mat = jnp.repeat(jnp.eye(EMB, dtype=jnp.float32), NUM_SPH, axis=0)

    return _tc_contract(g, sph, w, b_mat, p_mat, t, tile)


# speedup vs baseline: 1.7438x; 1.0759x over previous
import functools

import jax
import jax.numpy as jnp
from jax import lax
from jax.experimental import pallas as pl
from jax.experimental.pallas import tpu as pltpu
from jax.experimental.pallas import tpu_sc as plsc

NUM_RADIAL = 128
NUM_SPH = 8
EMB = 32
OUT_COLS = NUM_SPH * EMB

_NC = 2
_NS = 16
_NW = _NC * _NS
_CHUNK = 128


def _sc_gather(table, idx, t_pad, nc0, nc1):
    mesh = plsc.VectorSubcoreMesh(core_axis_name="c", subcore_axis_name="s")

    @functools.partial(
        pl.kernel,
        mesh=mesh,
        out_type=jax.ShapeDtypeStruct((t_pad, NUM_RADIAL), jnp.float32),
        scratch_types=[
            pltpu.VMEM((_CHUNK,), jnp.int32),
            pltpu.VMEM((_CHUNK,), jnp.int32),
            pltpu.VMEM((_CHUNK, NUM_RADIAL), jnp.float32),
            pltpu.VMEM((_CHUNK, NUM_RADIAL), jnp.float32),
            pltpu.SemaphoreType.DMA,
            pltpu.SemaphoreType.DMA,
            pltpu.SemaphoreType.DMA,
            pltpu.SemaphoreType.DMA,
            pltpu.SemaphoreType.DMA,
            pltpu.SemaphoreType.DMA,
        ],
    )
    def k(table_hbm, idx_hbm, out_hbm, idx0, idx1, rows0, rows1,
          g0, g1, w0, w1, i0, i1):
        c_ax = lax.axis_index("c")
        s_ax = lax.axis_index("s")
        my_n = jnp.where(c_ax == 0, nc0, nc1)
        base = jnp.where(c_ax == 0, s_ax * nc0, _NS * nc0 + s_ax * nc1)

        def off(c):
            return (base + c) * _CHUNK

        def fetch_idx(c, idxb, isem):
            pltpu.async_copy(idx_hbm.at[pl.ds(off(c), _CHUNK)], idxb, isem)

        def do_chunk(c, idxb, rowsb, gsem, wsem, isem,
                     drain_first, fetch_next):
            pltpu.make_async_copy(
                idx_hbm.at[pl.ds(off(c), _CHUNK)], idxb, isem).wait()
            if drain_first:
                pltpu.make_async_copy(
                    rowsb, out_hbm.at[pl.ds(off(c), _CHUNK)], wsem).wait()
            pltpu.async_copy(table_hbm.at[idxb], rowsb, gsem).wait()
            pltpu.async_copy(rowsb, out_hbm.at[pl.ds(off(c), _CHUNK)], wsem)
            if fetch_next:
                fetch_idx(c + 2, idxb, isem)

        fetch_idx(0, idx0, i0)
        fetch_idx(1, idx1, i1)
        do_chunk(0, idx0, rows0, g0, w0, i0, False, True)
        do_chunk(1, idx1, rows1, g1, w1, i1, False, True)

        def body(j, carry):
            do_chunk(2 * j, idx0, rows0, g0, w0, i0, True, True)
            do_chunk(2 * j + 1, idx1, rows1, g1, w1, i1, True, True)
            return carry

        lax.fori_loop(1, my_n // 2 - 1, body, 0, unroll=False)

        do_chunk(my_n - 2, idx0, rows0, g0, w0, i0, True, False)
        do_chunk(my_n - 1, idx1, rows1, g1, w1, i1, True, False)
        pltpu.make_async_copy(
            rows0, out_hbm.at[pl.ds(off(my_n - 2), _CHUNK)], w0).wait()
        pltpu.make_async_copy(
            rows1, out_hbm.at[pl.ds(off(my_n - 1), _CHUNK)], w1).wait()

    return k(table, idx)


def _tc_contract(g, sph, w, b_mat, p_mat, t, tile):
    def body(g_ref, s_ref, w_ref, b_ref, p_ref, o_ref):
        h = jnp.dot(g_ref[...], w_ref[...], preferred_element_type=jnp.float32)
        srep = jnp.dot(s_ref[...], b_ref[...], preferred_element_type=jnp.float32)
        o_ref[...] = jnp.dot(h * srep, p_ref[...],
                             preferred_element_type=jnp.float32)

    return pl.pallas_call(
        body,
        grid=(t // tile,),
        in_specs=[
            pl.BlockSpec((tile, NUM_RADIAL), lambda i: (i, 0)),
            pl.BlockSpec((tile, NUM_SPH), lambda i: (i, 0)),
            pl.BlockSpec((NUM_RADIAL, OUT_COLS), lambda i: (0, 0)),
            pl.BlockSpec((NUM_SPH, OUT_COLS), lambda i: (0, 0)),
            pl.BlockSpec((OUT_COLS, EMB), lambda i: (0, 0)),
        ],
        out_specs=pl.BlockSpec((tile, EMB), lambda i: (i, 0)),
        out_shape=jax.ShapeDtypeStruct((t, EMB), jnp.float32),
    )(g, sph, w, b_mat, p_mat)


def kernel(rbf, sph, idx_sph, weight):
    t = idx_sph.shape[0]
    tile = 4000
    nchunks = -(-t // (_NW * _CHUNK))
    nchunks += nchunks % 2
    t_pad = _NW * nchunks * _CHUNK
    nc0 = max(4, (2 * nchunks * 13 // 20) // 2 * 2)
    nc1 = 2 * nchunks - nc0

    idx_pad = jnp.zeros((t_pad,), jnp.int32).at[:t].set(idx_sph)

    g = _sc_gather(rbf, idx_pad, t_pad, nc0, nc1)

    w = weight.reshape(NUM_RADIAL, OUT_COLS)
    b_mat = jnp.tile(jnp.eye(NUM_SPH, dtype=jnp.float32), (1, EMB))
    p_mat = jnp.repeat(jnp.eye(EMB, dtype=jnp.float32), NUM_SPH, axis=0)

    return _tc_contract(g, sph, w, b_mat, p_mat, t, tile)


# speedup vs baseline: 1.7911x; 1.0271x over previous
import functools

import jax
import jax.numpy as jnp
from jax import lax
from jax.experimental import pallas as pl
from jax.experimental.pallas import tpu as pltpu
from jax.experimental.pallas import tpu_sc as plsc

NUM_RADIAL = 128
NUM_SPH = 8
EMB = 32
OUT_COLS = NUM_SPH * EMB

_NC = 2
_NS = 16
_NW = _NC * _NS
_CHUNK = 128


def _sc_gather(table, idx, t_pad, nc0, nc1):
    mesh = plsc.VectorSubcoreMesh(core_axis_name="c", subcore_axis_name="s")

    @functools.partial(
        pl.kernel,
        mesh=mesh,
        out_type=jax.ShapeDtypeStruct((t_pad, NUM_RADIAL), jnp.float32),
        scratch_types=[
            pltpu.VMEM((_CHUNK,), jnp.int32),
            pltpu.VMEM((_CHUNK,), jnp.int32),
            pltpu.VMEM((_CHUNK, NUM_RADIAL), jnp.float32),
            pltpu.VMEM((_CHUNK, NUM_RADIAL), jnp.float32),
            pltpu.SemaphoreType.DMA,
            pltpu.SemaphoreType.DMA,
            pltpu.SemaphoreType.DMA,
            pltpu.SemaphoreType.DMA,
            pltpu.SemaphoreType.DMA,
            pltpu.SemaphoreType.DMA,
        ],
    )
    def k(table_hbm, idx_hbm, out_hbm, idx0, idx1, rows0, rows1,
          g0, g1, w0, w1, i0, i1):
        c_ax = lax.axis_index("c")
        s_ax = lax.axis_index("s")
        my_n = jnp.where(c_ax == 0, nc0, nc1)
        base = jnp.where(c_ax == 0, s_ax * nc0, _NS * nc0 + s_ax * nc1)

        def off(c):
            return (base + c) * _CHUNK

        def fetch_idx(c, idxb, isem):
            pltpu.async_copy(idx_hbm.at[pl.ds(off(c), _CHUNK)], idxb, isem)

        def do_chunk(c, idxb, rowsb, gsem, wsem, isem,
                     drain_first, fetch_next):
            pltpu.make_async_copy(
                idx_hbm.at[pl.ds(off(c), _CHUNK)], idxb, isem).wait()
            if drain_first:
                pltpu.make_async_copy(
                    rowsb, out_hbm.at[pl.ds(off(c), _CHUNK)], wsem).wait()
            pltpu.async_copy(table_hbm.at[idxb], rowsb, gsem).wait()
            pltpu.async_copy(rowsb, out_hbm.at[pl.ds(off(c), _CHUNK)], wsem)
            if fetch_next:
                fetch_idx(c + 2, idxb, isem)

        fetch_idx(0, idx0, i0)
        fetch_idx(1, idx1, i1)
        do_chunk(0, idx0, rows0, g0, w0, i0, False, True)
        do_chunk(1, idx1, rows1, g1, w1, i1, False, True)

        def body(j, carry):
            do_chunk(2 * j, idx0, rows0, g0, w0, i0, True, True)
            do_chunk(2 * j + 1, idx1, rows1, g1, w1, i1, True, True)
            return carry

        lax.fori_loop(1, my_n // 2 - 1, body, 0, unroll=False)

        do_chunk(my_n - 2, idx0, rows0, g0, w0, i0, True, False)
        do_chunk(my_n - 1, idx1, rows1, g1, w1, i1, True, False)
        pltpu.make_async_copy(
            rows0, out_hbm.at[pl.ds(off(my_n - 2), _CHUNK)], w0).wait()
        pltpu.make_async_copy(
            rows1, out_hbm.at[pl.ds(off(my_n - 1), _CHUNK)], w1).wait()

    return k(table, idx)


def _tc_contract(g, sph, w, b_mat, p_mat, t, tile):
    def body(g_ref, s_ref, w_ref, b_ref, p_ref, o_ref):
        h = jnp.dot(g_ref[...], w_ref[...], preferred_element_type=jnp.float32)
        srep = jnp.dot(s_ref[...], b_ref[...], preferred_element_type=jnp.float32)
        o_ref[...] = jnp.dot(h * srep, p_ref[...],
                             preferred_element_type=jnp.float32)

    return pl.pallas_call(
        body,
        grid=(t // tile,),
        in_specs=[
            pl.BlockSpec((tile, NUM_RADIAL), lambda i: (i, 0)),
            pl.BlockSpec((tile, NUM_SPH), lambda i: (i, 0)),
            pl.BlockSpec((NUM_RADIAL, OUT_COLS), lambda i: (0, 0)),
            pl.BlockSpec((NUM_SPH, OUT_COLS), lambda i: (0, 0)),
            pl.BlockSpec((OUT_COLS, EMB), lambda i: (0, 0)),
        ],
        out_specs=pl.BlockSpec((tile, EMB), lambda i: (i, 0)),
        out_shape=jax.ShapeDtypeStruct((t, EMB), jnp.float32),
    )(g, sph, w, b_mat, p_mat)


def kernel(rbf, sph, idx_sph, weight):
    t = idx_sph.shape[0]
    tile = 6000
    nchunks = -(-t // (_NW * _CHUNK))
    nchunks += nchunks % 2
    t_pad = _NW * nchunks * _CHUNK
    nc0 = max(4, (2 * nchunks * 13 // 20) // 2 * 2)
    nc1 = 2 * nchunks - nc0

    idx_pad = jnp.zeros((t_pad,), jnp.int32).at[:t].set(idx_sph)

    g = _sc_gather(rbf, idx_pad, t_pad, nc0, nc1)

    w = weight.reshape(NUM_RADIAL, OUT_COLS)
    b_mat = jnp.tile(jnp.eye(NUM_SPH, dtype=jnp.float32), (1, EMB))
    p_mat = jnp.repeat(jnp.eye(EMB, dtype=jnp.float32), NUM_SPH, axis=0)

    return _tc_contract(g, sph, w, b_mat, p_mat, t, tile)


# speedup vs baseline: 1.8270x; 1.0201x over previous
import functools

import jax
import jax.numpy as jnp
from jax import lax
from jax.experimental import pallas as pl
from jax.experimental.pallas import tpu as pltpu
from jax.experimental.pallas import tpu_sc as plsc

NUM_RADIAL = 128
NUM_SPH = 8
EMB = 32
OUT_COLS = NUM_SPH * EMB

_NC = 2
_NS = 16
_NW = _NC * _NS
_CHUNK = 128


def _sc_gather(table, idx, t_pad, nc0, nc1):
    mesh = plsc.VectorSubcoreMesh(core_axis_name="c", subcore_axis_name="s")

    @functools.partial(
        pl.kernel,
        mesh=mesh,
        out_type=jax.ShapeDtypeStruct((t_pad, NUM_RADIAL), jnp.float32),
        scratch_types=[
            pltpu.VMEM((_CHUNK,), jnp.int32),
            pltpu.VMEM((_CHUNK,), jnp.int32),
            pltpu.VMEM((_CHUNK, NUM_RADIAL), jnp.float32),
            pltpu.VMEM((_CHUNK, NUM_RADIAL), jnp.float32),
            pltpu.SemaphoreType.DMA,
            pltpu.SemaphoreType.DMA,
            pltpu.SemaphoreType.DMA,
            pltpu.SemaphoreType.DMA,
            pltpu.SemaphoreType.DMA,
            pltpu.SemaphoreType.DMA,
        ],
    )
    def k(table_hbm, idx_hbm, out_hbm, idx0, idx1, rows0, rows1,
          g0, g1, w0, w1, i0, i1):
        c_ax = lax.axis_index("c")
        s_ax = lax.axis_index("s")
        my_n = jnp.where(c_ax == 0, nc0, nc1)
        base = jnp.where(c_ax == 0, s_ax * nc0, _NS * nc0 + s_ax * nc1)

        def off(c):
            return (base + c) * _CHUNK

        def fetch_idx(c, idxb, isem):
            pltpu.async_copy(idx_hbm.at[pl.ds(off(c), _CHUNK)], idxb, isem)

        def do_chunk(c, idxb, rowsb, gsem, wsem, isem,
                     drain_first, fetch_next):
            pltpu.make_async_copy(
                idx_hbm.at[pl.ds(off(c), _CHUNK)], idxb, isem).wait()
            if drain_first:
                pltpu.make_async_copy(
                    rowsb, out_hbm.at[pl.ds(off(c), _CHUNK)], wsem).wait()
            pltpu.async_copy(table_hbm.at[idxb], rowsb, gsem).wait()
            pltpu.async_copy(rowsb, out_hbm.at[pl.ds(off(c), _CHUNK)], wsem)
            if fetch_next:
                fetch_idx(c + 2, idxb, isem)

        fetch_idx(0, idx0, i0)
        fetch_idx(1, idx1, i1)
        do_chunk(0, idx0, rows0, g0, w0, i0, False, True)
        do_chunk(1, idx1, rows1, g1, w1, i1, False, True)

        def body(j, carry):
            do_chunk(2 * j, idx0, rows0, g0, w0, i0, True, True)
            do_chunk(2 * j + 1, idx1, rows1, g1, w1, i1, True, True)
            return carry

        lax.fori_loop(1, my_n // 2 - 1, body, 0, unroll=False)

        do_chunk(my_n - 2, idx0, rows0, g0, w0, i0, True, False)
        do_chunk(my_n - 1, idx1, rows1, g1, w1, i1, True, False)
        pltpu.make_async_copy(
            rows0, out_hbm.at[pl.ds(off(my_n - 2), _CHUNK)], w0).wait()
        pltpu.make_async_copy(
            rows1, out_hbm.at[pl.ds(off(my_n - 1), _CHUNK)], w1).wait()

    return k(table, idx)


def _tc_contract(g, sph, w, b_mat, p_mat, t, tile):
    def body(g_ref, s_ref, w_ref, b_ref, p_ref, o_ref):
        h = jnp.dot(g_ref[...], w_ref[...], preferred_element_type=jnp.float32)
        srep = jnp.dot(s_ref[...], b_ref[...], preferred_element_type=jnp.float32)
        o_ref[...] = jnp.dot(h * srep, p_ref[...],
                             preferred_element_type=jnp.float32)

    return pl.pallas_call(
        body,
        grid=(t // tile,),
        in_specs=[
            pl.BlockSpec((tile, NUM_RADIAL), lambda i: (i, 0)),
            pl.BlockSpec((tile, NUM_SPH), lambda i: (i, 0)),
            pl.BlockSpec((NUM_RADIAL, OUT_COLS), lambda i: (0, 0)),
            pl.BlockSpec((NUM_SPH, OUT_COLS), lambda i: (0, 0)),
            pl.BlockSpec((OUT_COLS, EMB), lambda i: (0, 0)),
        ],
        out_specs=pl.BlockSpec((tile, EMB), lambda i: (i, 0)),
        out_shape=jax.ShapeDtypeStruct((t, EMB), jnp.float32),
    )(g, sph, w, b_mat, p_mat)


def kernel(rbf, sph, idx_sph, weight):
    t = idx_sph.shape[0]
    tile = 10000
    nchunks = -(-t // (_NW * _CHUNK))
    nchunks += nchunks % 2
    t_pad = _NW * nchunks * _CHUNK
    nc0 = max(4, (2 * nchunks * 13 // 20) // 2 * 2)
    nc1 = 2 * nchunks - nc0

    idx_pad = jnp.zeros((t_pad,), jnp.int32).at[:t].set(idx_sph)

    g = _sc_gather(rbf, idx_pad, t_pad, nc0, nc1)

    w = weight.reshape(NUM_RADIAL, OUT_COLS)
    b_mat = jnp.tile(jnp.eye(NUM_SPH, dtype=jnp.float32), (1, EMB))
    p_mat = jnp.repeat(jnp.eye(EMB, dtype=jnp.float32), NUM_SPH, axis=0)

    return _tc_contract(g, sph, w, b_mat, p_mat, t, tile)
